# Initial kernel scaffold; baseline (speedup 1.0000x reference)
import jax
import jax.numpy as jnp
from jax.experimental import pallas as pl


def kernel(edge_index, edge_weight):
    raise NotImplementedError("write your pallas kernel here")


# speedup vs baseline: 396.9855x; 396.9855x over previous
import functools

import jax
import jax.numpy as jnp
from jax import lax
from jax.experimental import pallas as pl
from jax.experimental.pallas import tpu as pltpu
from jax.experimental.pallas import tpu_sc as plsc

_N = 100000
_E = 6400000

_NCORES = 2
_NTILES = 16
_NWORK = _NCORES * _NTILES
_NP = 100352
_TSLICE = _NP // _NTILES
_W = 4000
_EPW = _E // _NWORK
_NWIN = _EPW // _W


def _sc_degree_partials(edge_index, edge_weight):
    mesh = plsc.VectorSubcoreMesh(core_axis_name="c", subcore_axis_name="s")

    @functools.partial(
        pl.kernel,
        out_type=(
            jax.ShapeDtypeStruct((_NCORES, _NP), jnp.float32),
            jax.ShapeDtypeStruct((_NCORES, _NP), jnp.float32),
        ),
        mesh=mesh,
        scratch_types=(
            pltpu.VMEM((_W,), jnp.int32),
            pltpu.VMEM((_W,), jnp.int32),
            pltpu.VMEM((_W,), jnp.float32),
            pltpu.VMEM((_TSLICE,), jnp.float32),
            pltpu.VMEM_SHARED((_NP,), jnp.float32),
            pltpu.VMEM_SHARED((_NP,), jnp.float32),
        ),
    )
    def k(ei_hbm, w_hbm, dout_hbm, din_hbm,
          row_v, col_v, w_v, zero_v, acc_out, acc_in):
        c = lax.axis_index("c")
        s = lax.axis_index("s")
        wid = c * _NTILES + s

        def zb(i, carry):
            zero_v[pl.ds(i * 16, 16)] = jnp.zeros((16,), jnp.float32)
            return carry
        lax.fori_loop(0, _TSLICE // 16, zb, 0)

        sl = pl.ds(s * _TSLICE, _TSLICE)
        pltpu.sync_copy(zero_v, acc_out.at[sl])
        pltpu.sync_copy(zero_v, acc_in.at[sl])
        plsc.subcore_barrier()

        base = wid * _EPW

        def win(i, carry):
            off = base + i * _W
            pltpu.sync_copy(ei_hbm.at[pl.ds(off, _W)], row_v)
            pltpu.sync_copy(ei_hbm.at[pl.ds(_E + off, _W)], col_v)
            pltpu.sync_copy(w_hbm.at[pl.ds(off, _W)], w_v)
            pltpu.sync_copy(w_v, acc_out.at[row_v], add=True)
            pltpu.sync_copy(w_v, acc_in.at[col_v], add=True)
            return carry
        lax.fori_loop(0, _NWIN, win, 0)

        plsc.subcore_barrier()
        pltpu.sync_copy(acc_out.at[sl], dout_hbm.at[c, sl])
        pltpu.sync_copy(acc_in.at[sl], din_hbm.at[c, sl])

    return k(edge_index.reshape(2 * _E), edge_weight)


def _tc_epilogue(dout2, din2):
    def body(dout_ref, din_ref, pout_ref, pin_ref, s_ref, c0_ref):
        dout = dout_ref[0] + dout_ref[1]
        din = din_ref[0] + din_ref[1]
        s_out = jnp.sum(dout)
        s_in = jnp.sum(din)
        pout = dout / s_out
        pin = din / s_in
        pout_ref[...] = pout
        pin_ref[...] = pin
        r = jnp.sum(
            jnp.where(dout > 0, dout * jnp.log(pin), 0.0)
            + jnp.where(din > 0, din * jnp.log(pout), 0.0)
        )
        s_ref[0, 0] = s_out
        c0_ref[0, 0] = -jnp.log(s_out) - r / s_out

    return pl.pallas_call(
        body,
        out_specs=(
            pl.BlockSpec(memory_space=pltpu.VMEM),
            pl.BlockSpec(memory_space=pltpu.VMEM),
            pl.BlockSpec(memory_space=pltpu.SMEM),
            pl.BlockSpec(memory_space=pltpu.SMEM),
        ),
        out_shape=(
            jax.ShapeDtypeStruct((784, 128), jnp.float32),
            jax.ShapeDtypeStruct((784, 128), jnp.float32),
            jax.ShapeDtypeStruct((1, 1), jnp.float32),
            jax.ShapeDtypeStruct((1, 1), jnp.float32),
        ),
    )(dout2, din2)


_ROWS = _E // 128
_BR = 2000
_GB = _ROWS // _BR


def _tc_main(w2d, s, c0):
    def body(w_ref, s_ref, c0_ref, puv_ref, mi_ref):
        i = pl.program_id(0)
        sval = s_ref[0, 0]
        w = w_ref[...]
        puv_ref[...] = w / sval
        part = jnp.sum(w * jnp.log(w))

        @pl.when(i == 0)
        def _():
            mi_ref[0, 0] = c0_ref[0, 0]

        mi_ref[0, 0] += part / sval

    return pl.pallas_call(
        body,
        grid=(_GB,),
        in_specs=[
            pl.BlockSpec((_BR, 128), lambda i: (i, 0)),
            pl.BlockSpec(memory_space=pltpu.SMEM),
            pl.BlockSpec(memory_space=pltpu.SMEM),
        ],
        out_specs=[
            pl.BlockSpec((_BR, 128), lambda i: (i, 0)),
            pl.BlockSpec(memory_space=pltpu.SMEM),
        ],
        out_shape=(
            jax.ShapeDtypeStruct((_ROWS, 128), jnp.float32),
            jax.ShapeDtypeStruct((1, 1), jnp.float32),
        ),
    )(w2d, s, c0)


def kernel(edge_index, edge_weight):
    dout_p, din_p = _sc_degree_partials(edge_index, edge_weight)
    pout_pad, pin_pad, s, c0 = _tc_epilogue(
        dout_p.reshape(_NCORES, 784, 128), din_p.reshape(_NCORES, 784, 128)
    )
    puv2d, mi = _tc_main(edge_weight.reshape(_ROWS, 128), s, c0)

    node_ids = jnp.arange(_N, dtype=jnp.int32)
    p_uv = puv2d.reshape(_E)
    p_u_out = pout_pad.reshape(_NP)[:_N]
    p_u_in = pin_pad.reshape(_NP)[:_N]
    return (node_ids, p_uv, p_u_out, p_u_in, mi.reshape(()))
